# baseline (speedup 1.0000x reference)
import functools

import jax
import jax.numpy as jnp
from jax import lax
from jax.experimental import pallas as pl
from jax.experimental.pallas import tpu as pltpu
from jax.experimental.pallas import tpu_sc as plsc

_EPS = 1e-5
_BN = 1.0 / (1.0 + _EPS) ** 0.5


def _lrelu(x, slope):
    return jnp.where(x > 0, x, slope * x)


def _in_mlp_body(h_ref, w_ref, b_ref, o_ref):
    x = _lrelu(h_ref[...] * _BN, 0.2)
    y = jnp.dot(x, w_ref[...], preferred_element_type=jnp.float32) + b_ref[...]
    o_ref[...] = _lrelu(y * _BN, 0.2)


def _tc_in_mlp(h, W, b):
    N, D = h.shape
    BLK = 1000
    return pl.pallas_call(
        _in_mlp_body,
        grid=(N // BLK,),
        in_specs=[
            pl.BlockSpec((BLK, D), lambda i: (i, 0)),
            pl.BlockSpec((D, D), lambda i: (0, 0)),
            pl.BlockSpec((1, D), lambda i: (0, 0)),
        ],
        out_specs=pl.BlockSpec((BLK, D), lambda i: (i, 0)),
        out_shape=jax.ShapeDtypeStruct((N, D), jnp.float32),
    )(h, W, b.reshape(1, D))


def _edge_mlp_body(he_ref, w_ref, b_ref, o_ref):
    x = _lrelu(he_ref[...] * _BN, 0.2)
    y = jnp.dot(x, w_ref[...], preferred_element_type=jnp.float32) + b_ref[...]
    o_ref[...] = _lrelu(y * _BN, 0.2)


def _tc_edge_mlp(h_e, W, b):
    E, DE = h_e.shape
    D = W.shape[1]
    BLK = 4000
    return pl.pallas_call(
        _edge_mlp_body,
        grid=(E // BLK,),
        in_specs=[
            pl.BlockSpec((BLK, DE), lambda i: (i, 0)),
            pl.BlockSpec((DE, D), lambda i: (0, 0)),
            pl.BlockSpec((1, D), lambda i: (0, 0)),
        ],
        out_specs=pl.BlockSpec((BLK, D), lambda i: (i, 0)),
        out_shape=jax.ShapeDtypeStruct((E, D), jnp.float32),
    )(h_e, W, b.reshape(1, D))


def _out_mlp_body(u_ref, h_ref, wl_ref, wo_ref, o_ref):
    x = _lrelu(u_ref[...] * _BN, 0.1)
    y = jnp.dot(x, wl_ref[...], preferred_element_type=jnp.float32)
    y = _lrelu(y * _BN, 0.1)
    z = jnp.dot(y, wo_ref[...], preferred_element_type=jnp.float32)
    o_ref[...] = z + h_ref[...]


def _tc_out_mlp(upd, h, W_lin, W_out):
    N, RD = upd.shape
    D = W_out.shape[1]
    BLK = 1000
    return pl.pallas_call(
        _out_mlp_body,
        grid=(N // BLK,),
        in_specs=[
            pl.BlockSpec((BLK, RD), lambda i: (i, 0)),
            pl.BlockSpec((BLK, D), lambda i: (i, 0)),
            pl.BlockSpec((RD, D), lambda i: (0, 0)),
            pl.BlockSpec((D, D), lambda i: (0, 0)),
        ],
        out_specs=pl.BlockSpec((BLK, D), lambda i: (i, 0)),
        out_shape=jax.ShapeDtypeStruct((N, D), jnp.float32),
    )(upd, h, W_lin, W_out)


_NC, _NS = 2, 16


@functools.lru_cache(maxsize=None)
def _make_sc_scatter(N, E, D, R):
    NR = N * R
    NB = 16
    BR = -(-NR // NB)
    BR += (-BR) % 128
    TRASH = BR
    EPT = E // _NS
    S = 4000
    NCH = EPT // S
    GC = 256
    CB = 4096
    SHARE = BR // _NS
    assert E % _NS == 0 and EPT % S == 0 and S % 16 == 0 and BR % _NS == 0

    mesh = plsc.VectorSubcoreMesh(
        core_axis_name="c", subcore_axis_name="s", num_cores=_NC, num_subcores=_NS
    )

    @functools.partial(
        pl.kernel,
        out_type=jax.ShapeDtypeStruct((NB * BR, D), jnp.float32),
        mesh=mesh,
        scratch_types=[
            pltpu.VMEM((S,), jnp.int32),
            pltpu.VMEM((S,), jnp.int32),
            pltpu.VMEM((S,), jnp.int32),
            pltpu.VMEM((CB,), jnp.int32),
            pltpu.VMEM((CB,), jnp.int32),
            pltpu.VMEM((CB,), jnp.int32),
            pltpu.VMEM((GC,), jnp.int32),
            pltpu.VMEM((GC, D), jnp.float32),
            pltpu.VMEM((GC, D), jnp.float32),
            pltpu.VMEM_SHARED((BR + 8, D), jnp.float32),
        ],
        compiler_params=pltpu.CompilerParams(needs_layout_passes=False),
    )
    def sc_kernel(hv_hbm, he_hbm, src_hbm, dst_hbm, rel_hbm, out_hbm,
                  dstv, relv, srcv, c_lidx, c_src, c_eid, sidx,
                  rows_a, rows_b, acc):
        cid = lax.axis_index("c")
        sid = lax.axis_index("s")
        ebase = sid * EPT

        zero16i = jnp.zeros((16,), jnp.int32)
        zero16f = jnp.zeros((16,), jnp.float32)
        trash16 = jnp.full((16,), TRASH, jnp.int32)
        iota16 = lax.iota(jnp.int32, 16)

        @pl.loop(0, CB, step=16)
        def _(i):
            c_src[pl.ds(i, 16)] = zero16i
            c_eid[pl.ds(i, 16)] = zero16i

        for bi in range(NB // _NC):
            b = bi * _NC + cid
            lo = b * BR

            @pl.loop(0, GC)
            def _(r):
                for c in range(D // 16):
                    rows_a[r, pl.ds(c * 16, 16)] = zero16f

            for z0 in range(0, SHARE, GC):
                zn = min(GC, SHARE - z0)
                pltpu.sync_copy(
                    rows_a.at[pl.ds(0, zn)], acc.at[pl.ds(sid * SHARE + z0, zn)]
                )

            @pl.when(sid == 0)
            def _():
                pltpu.sync_copy(rows_a.at[pl.ds(0, 8)], acc.at[pl.ds(BR, 8)])

            plsc.subcore_barrier()

            for ch in range(NCH):
                cbase = ebase + ch * S
                pltpu.sync_copy(dst_hbm.at[pl.ds(cbase, S)], dstv)
                pltpu.sync_copy(rel_hbm.at[pl.ds(cbase, S)], relv)
                pltpu.sync_copy(src_hbm.at[pl.ds(cbase, S)], srcv)

                @pl.loop(0, CB, step=16)
                def _(i):
                    c_lidx[pl.ds(i, 16)] = trash16

                def scan_body(v, cnt):
                    sl = pl.ds(v * 16, 16)
                    dst16 = dstv[sl]
                    rel16 = relv[sl]
                    src16 = srcv[sl]
                    idx16 = dst16 * R + rel16
                    m = (idx16 >= lo) & (idx16 < lo + BR)
                    lidx16 = idx16 - lo
                    eid16 = cbase + v * 16 + iota16
                    mi = m.astype(jnp.int32)
                    run = plsc.cumsum(mi)
                    pos16 = cnt + run - 1
                    plsc.store_scatter(c_lidx, [pos16], lidx16, mask=m)
                    plsc.store_scatter(c_src, [pos16], src16, mask=m)
                    plsc.store_scatter(c_eid, [pos16], eid16, mask=m)
                    return cnt + jnp.sum(mi)

                cnt = lax.fori_loop(0, S // 16, scan_body, jnp.int32(0))
                nsub = (cnt + GC - 1) // GC

                def sub_body(j, _):
                    off = j * GC
                    pltpu.sync_copy(hv_hbm.at[c_src.at[pl.ds(off, GC)]], rows_a)
                    pltpu.sync_copy(he_hbm.at[c_eid.at[pl.ds(off, GC)]], rows_b)

                    @pl.loop(0, GC, step=16)
                    def _(i):
                        sidx[pl.ds(i, 16)] = c_lidx[pl.ds(off + i, 16)]

                    pltpu.sync_copy(rows_a, acc.at[sidx], add=True)
                    pltpu.sync_copy(rows_b, acc.at[sidx], add=True)
                    return 0

                lax.fori_loop(0, nsub, sub_body, 0)

            plsc.subcore_barrier()
            pltpu.sync_copy(
                acc.at[pl.ds(sid * SHARE, SHARE)],
                out_hbm.at[pl.ds(lo + sid * SHARE, SHARE)],
            )

    return sc_kernel


def kernel(h_v, edge_index, h_e, W_in, b_in, W_edge, b_edge, W_lin, W_out):
    N, D = h_v.shape
    E, DE = h_e.shape
    L = W_in.shape[0]
    R = W_lin.shape[1] // D
    NR = N * R

    sc_scatter = _make_sc_scatter(N, E, D, R)
    ei = edge_index.astype(jnp.int32)
    src, dst, rel = ei[0], ei[1], ei[2]

    h = h_v
    for l in range(L):
        hv = _tc_in_mlp(h, W_in[l], b_in[l])
        he = _tc_edge_mlp(h_e, W_edge[l], b_edge[l])
        upd_full = sc_scatter(hv, he, src, dst, rel)
        upd = upd_full[:NR].reshape(N, R * D)
        h = _tc_out_mlp(upd, h, W_lin[l], W_out[l])
    return h


# speedup vs baseline: 1.0398x; 1.0398x over previous
import functools

import jax
import jax.numpy as jnp
from jax import lax
from jax.experimental import pallas as pl
from jax.experimental.pallas import tpu as pltpu
from jax.experimental.pallas import tpu_sc as plsc

_EPS = 1e-5
_BN = 1.0 / (1.0 + _EPS) ** 0.5


def _lrelu(x, slope):
    return jnp.where(x > 0, x, slope * x)


def _in_mlp_body(h_ref, w_ref, b_ref, o_ref):
    x = _lrelu(h_ref[...] * _BN, 0.2)
    y = jnp.dot(x, w_ref[...], preferred_element_type=jnp.float32) + b_ref[...]
    o_ref[...] = _lrelu(y * _BN, 0.2)


def _tc_in_mlp(h, W, b):
    N, D = h.shape
    BLK = 1000
    return pl.pallas_call(
        _in_mlp_body,
        grid=(N // BLK,),
        in_specs=[
            pl.BlockSpec((BLK, D), lambda i: (i, 0)),
            pl.BlockSpec((D, D), lambda i: (0, 0)),
            pl.BlockSpec((1, D), lambda i: (0, 0)),
        ],
        out_specs=pl.BlockSpec((BLK, D), lambda i: (i, 0)),
        out_shape=jax.ShapeDtypeStruct((N, D), jnp.float32),
    )(h, W, b.reshape(1, D))


def _edge_mlp_body(he_ref, w_ref, b_ref, o_ref):
    x = _lrelu(he_ref[...] * _BN, 0.2)
    y = jnp.dot(x, w_ref[...], preferred_element_type=jnp.float32) + b_ref[...]
    o_ref[...] = _lrelu(y * _BN, 0.2)


def _tc_edge_mlp(h_e, W, b):
    E, DE = h_e.shape
    D = W.shape[1]
    BLK = 4000
    return pl.pallas_call(
        _edge_mlp_body,
        grid=(E // BLK,),
        in_specs=[
            pl.BlockSpec((BLK, DE), lambda i: (i, 0)),
            pl.BlockSpec((DE, D), lambda i: (0, 0)),
            pl.BlockSpec((1, D), lambda i: (0, 0)),
        ],
        out_specs=pl.BlockSpec((BLK, D), lambda i: (i, 0)),
        out_shape=jax.ShapeDtypeStruct((E, D), jnp.float32),
    )(h_e, W, b.reshape(1, D))


def _out_mlp_body(u_ref, h_ref, wl_ref, wo_ref, o_ref):
    x = _lrelu(u_ref[...] * _BN, 0.1)
    y = jnp.dot(x, wl_ref[...], preferred_element_type=jnp.float32)
    y = _lrelu(y * _BN, 0.1)
    z = jnp.dot(y, wo_ref[...], preferred_element_type=jnp.float32)
    o_ref[...] = z + h_ref[...]


def _tc_out_mlp(upd, h, W_lin, W_out):
    N, RD = upd.shape
    D = W_out.shape[1]
    BLK = 1000
    return pl.pallas_call(
        _out_mlp_body,
        grid=(N // BLK,),
        in_specs=[
            pl.BlockSpec((BLK, RD), lambda i: (i, 0)),
            pl.BlockSpec((BLK, D), lambda i: (i, 0)),
            pl.BlockSpec((RD, D), lambda i: (0, 0)),
            pl.BlockSpec((D, D), lambda i: (0, 0)),
        ],
        out_specs=pl.BlockSpec((BLK, D), lambda i: (i, 0)),
        out_shape=jax.ShapeDtypeStruct((N, D), jnp.float32),
    )(upd, h, W_lin, W_out)


_NC, _NS = 2, 16


@functools.lru_cache(maxsize=None)
def _make_sc_scatter(N, E, D, R):
    NR = N * R
    NB = 16
    BR = -(-NR // NB)
    BR += (-BR) % 128
    TRASH = BR
    EPT = E // _NS
    S = 4000
    NCH = EPT // S
    GC = 128
    CB = 4096
    SHARE = BR // _NS
    assert E % _NS == 0 and EPT % S == 0 and S % 16 == 0 and BR % _NS == 0

    mesh = plsc.VectorSubcoreMesh(
        core_axis_name="c", subcore_axis_name="s", num_cores=_NC, num_subcores=_NS
    )

    @functools.partial(
        pl.kernel,
        out_type=jax.ShapeDtypeStruct((NB * BR, D), jnp.float32),
        mesh=mesh,
        scratch_types=[
            pltpu.VMEM((S,), jnp.int32),
            pltpu.VMEM((S,), jnp.int32),
            pltpu.VMEM((S,), jnp.int32),
            pltpu.VMEM((CB,), jnp.int32),
            pltpu.VMEM((CB,), jnp.int32),
            pltpu.VMEM((CB,), jnp.int32),
            pltpu.VMEM((GC,), jnp.int32),
            pltpu.VMEM((GC,), jnp.int32),
            pltpu.VMEM((GC, D), jnp.float32),
            pltpu.VMEM((GC, D), jnp.float32),
            pltpu.VMEM((GC, D), jnp.float32),
            pltpu.VMEM((GC, D), jnp.float32),
            pltpu.VMEM_SHARED((BR + 8, D), jnp.float32),
            pltpu.SemaphoreType.DMA,
            pltpu.SemaphoreType.DMA,
            pltpu.SemaphoreType.DMA,
            pltpu.SemaphoreType.DMA,
            pltpu.SemaphoreType.DMA,
        ],
        compiler_params=pltpu.CompilerParams(needs_layout_passes=False),
    )
    def sc_kernel(hv_hbm, he_hbm, src_hbm, dst_hbm, rel_hbm, out_hbm,
                  dstv, relv, srcv, c_lidx, c_src, c_eid, sidx0, sidx1,
                  ra0, rb0, ra1, rb1, acc, si, sg0, sg1, ss0, ss1):
        cid = lax.axis_index("c")
        sid = lax.axis_index("s")
        ebase = sid * EPT

        zero16i = jnp.zeros((16,), jnp.int32)
        zero16f = jnp.zeros((16,), jnp.float32)
        trash16 = jnp.full((16,), TRASH, jnp.int32)
        iota16 = lax.iota(jnp.int32, 16)

        @pl.loop(0, CB, step=16)
        def _(i):
            c_src[pl.ds(i, 16)] = zero16i
            c_eid[pl.ds(i, 16)] = zero16i

        @pl.loop(0, NB // _NC)
        def _(bi):
            b = bi * _NC + cid
            lo = b * BR

            @pl.loop(0, GC)
            def _(r):
                for c in range(D // 16):
                    ra0[r, pl.ds(c * 16, 16)] = zero16f

            for z0 in range(0, SHARE, GC):
                zn = min(GC, SHARE - z0)
                pltpu.sync_copy(
                    ra0.at[pl.ds(0, zn)], acc.at[pl.ds(sid * SHARE + z0, zn)]
                )

            @pl.when(sid == 0)
            def _():
                pltpu.sync_copy(ra0.at[pl.ds(0, 8)], acc.at[pl.ds(BR, 8)])

            plsc.subcore_barrier()

            @pl.loop(0, NCH)
            def _(ch):
                cbase = ebase + ch * S
                ld = pltpu.async_copy(dst_hbm.at[pl.ds(cbase, S)], dstv, si)
                lr = pltpu.async_copy(rel_hbm.at[pl.ds(cbase, S)], relv, si)
                ls = pltpu.async_copy(src_hbm.at[pl.ds(cbase, S)], srcv, si)

                @pl.loop(0, CB, step=16)
                def _(i):
                    c_lidx[pl.ds(i, 16)] = trash16

                ld.wait()
                lr.wait()
                ls.wait()

                def scan_body(v, cnt):
                    sl = pl.ds(v * 16, 16)
                    dst16 = dstv[sl]
                    rel16 = relv[sl]
                    src16 = srcv[sl]
                    idx16 = dst16 * R + rel16
                    m = (idx16 >= lo) & (idx16 < lo + BR)
                    lidx16 = idx16 - lo
                    eid16 = cbase + v * 16 + iota16
                    mi = m.astype(jnp.int32)
                    run = plsc.cumsum(mi)
                    pos16 = cnt + run - 1
                    plsc.store_scatter(c_lidx, [pos16], lidx16, mask=m)
                    plsc.store_scatter(c_src, [pos16], src16, mask=m)
                    plsc.store_scatter(c_eid, [pos16], eid16, mask=m)
                    return cnt + jnp.sum(mi)

                cnt = lax.fori_loop(0, S // 16, scan_body, jnp.int32(0))
                npairs = (cnt + 2 * GC - 1) // (2 * GC)

                def pair_body(k, _):
                    off0 = k * (2 * GC)
                    off1 = off0 + GC

                    @pl.loop(0, GC, step=16)
                    def _(i):
                        sidx0[pl.ds(i, 16)] = c_lidx[pl.ds(off0 + i, 16)]
                        sidx1[pl.ds(i, 16)] = c_lidx[pl.ds(off1 + i, 16)]

                    h0 = pltpu.async_copy(
                        hv_hbm.at[c_src.at[pl.ds(off0, GC)]], ra0, sg0)
                    e0 = pltpu.async_copy(
                        he_hbm.at[c_eid.at[pl.ds(off0, GC)]], rb0, sg0)
                    h1 = pltpu.async_copy(
                        hv_hbm.at[c_src.at[pl.ds(off1, GC)]], ra1, sg1)
                    e1 = pltpu.async_copy(
                        he_hbm.at[c_eid.at[pl.ds(off1, GC)]], rb1, sg1)

                    h0.wait()
                    e0.wait()

                    @pl.loop(0, GC)
                    def _(r):
                        for c in range(D // 16):
                            cs = pl.ds(c * 16, 16)
                            ra0[r, cs] = ra0[r, cs] + rb0[r, cs]

                    s0 = pltpu.async_copy(ra0, acc.at[sidx0], ss0, add=True)

                    h1.wait()
                    e1.wait()

                    @pl.loop(0, GC)
                    def _(r):
                        for c in range(D // 16):
                            cs = pl.ds(c * 16, 16)
                            ra1[r, cs] = ra1[r, cs] + rb1[r, cs]

                    s1 = pltpu.async_copy(ra1, acc.at[sidx1], ss1, add=True)

                    s0.wait()
                    s1.wait()
                    return 0

                lax.fori_loop(0, npairs, pair_body, 0)

            plsc.subcore_barrier()
            pltpu.sync_copy(
                acc.at[pl.ds(sid * SHARE, SHARE)],
                out_hbm.at[pl.ds(lo + sid * SHARE, SHARE)],
            )

    return sc_kernel


def kernel(h_v, edge_index, h_e, W_in, b_in, W_edge, b_edge, W_lin, W_out):
    N, D = h_v.shape
    E, DE = h_e.shape
    L = W_in.shape[0]
    R = W_lin.shape[1] // D
    NR = N * R

    sc_scatter = _make_sc_scatter(N, E, D, R)
    ei = edge_index.astype(jnp.int32)
    src, dst, rel = ei[0], ei[1], ei[2]

    h = h_v
    for l in range(L):
        hv = _tc_in_mlp(h, W_in[l], b_in[l])
        he = _tc_edge_mlp(h_e, W_edge[l], b_edge[l])
        upd_full = sc_scatter(hv, he, src, dst, rel)
        upd = upd_full[:NR].reshape(N, R * D)
        h = _tc_out_mlp(upd, h, W_lin[l], W_out[l])
    return h


# speedup vs baseline: 4.4923x; 4.3204x over previous
import functools

import jax
import jax.numpy as jnp
from jax import lax
from jax.experimental import pallas as pl
from jax.experimental.pallas import tpu as pltpu
from jax.experimental.pallas import tpu_sc as plsc

_EPS = 1e-5
_BN = 1.0 / (1.0 + _EPS) ** 0.5


def _lrelu(x, slope):
    return jnp.where(x > 0, x, slope * x)


def _in_mlp_body(h_ref, w_ref, b_ref, o_ref):
    x = _lrelu(h_ref[...] * _BN, 0.2)
    y = jnp.dot(x, w_ref[...], preferred_element_type=jnp.float32) + b_ref[...]
    o_ref[...] = _lrelu(y * _BN, 0.2)


def _tc_in_mlp(h, W, b):
    N, D = h.shape
    BLK = 1000
    return pl.pallas_call(
        _in_mlp_body,
        grid=(N // BLK,),
        in_specs=[
            pl.BlockSpec((BLK, D), lambda i: (i, 0)),
            pl.BlockSpec((D, D), lambda i: (0, 0)),
            pl.BlockSpec((1, D), lambda i: (0, 0)),
        ],
        out_specs=pl.BlockSpec((BLK, D), lambda i: (i, 0)),
        out_shape=jax.ShapeDtypeStruct((N, D), jnp.float32),
    )(h, W, b.reshape(1, D))


def _edge_mlp_body(he_ref, w_ref, b_ref, o_ref):
    x = _lrelu(he_ref[...] * _BN, 0.2)
    y = jnp.dot(x, w_ref[...], preferred_element_type=jnp.float32) + b_ref[...]
    o_ref[...] = _lrelu(y * _BN, 0.2)


def _tc_edge_mlp(h_e, W, b):
    E, DE = h_e.shape
    D = W.shape[1]
    BLK = 4000
    return pl.pallas_call(
        _edge_mlp_body,
        grid=(E // BLK,),
        in_specs=[
            pl.BlockSpec((BLK, DE), lambda i: (i, 0)),
            pl.BlockSpec((DE, D), lambda i: (0, 0)),
            pl.BlockSpec((1, D), lambda i: (0, 0)),
        ],
        out_specs=pl.BlockSpec((BLK, D), lambda i: (i, 0)),
        out_shape=jax.ShapeDtypeStruct((E, D), jnp.float32),
    )(h_e, W, b.reshape(1, D))


def _out_mlp_body(u_ref, h_ref, wl_ref, wo_ref, o_ref):
    x = _lrelu(u_ref[...] * _BN, 0.1)
    y = jnp.dot(x, wl_ref[...], preferred_element_type=jnp.float32)
    y = _lrelu(y * _BN, 0.1)
    z = jnp.dot(y, wo_ref[...], preferred_element_type=jnp.float32)
    o_ref[...] = z + h_ref[...]


def _tc_out_mlp(upd, h, W_lin, W_out):
    N, RD = upd.shape
    D = W_out.shape[1]
    BLK = 1000
    return pl.pallas_call(
        _out_mlp_body,
        grid=(N // BLK,),
        in_specs=[
            pl.BlockSpec((BLK, RD), lambda i: (i, 0)),
            pl.BlockSpec((BLK, D), lambda i: (i, 0)),
            pl.BlockSpec((RD, D), lambda i: (0, 0)),
            pl.BlockSpec((D, D), lambda i: (0, 0)),
        ],
        out_specs=pl.BlockSpec((BLK, D), lambda i: (i, 0)),
        out_shape=jax.ShapeDtypeStruct((N, D), jnp.float32),
    )(upd, h, W_lin, W_out)


_NC, _NS = 2, 16


@functools.lru_cache(maxsize=None)
def _make_sc_kernels(N, E, D, R):
    NR = N * R
    BR = 4096
    NB = -(-NR // BR)
    NBC = NB // _NC
    TRASH = BR
    EPT = E // _NS
    S = 4000
    NCH = EPT // S
    GC = 128
    CAP = ((EPT + 143) // GC + 1) * GC
    SEGS = _NC * _NS * NBC
    TOTSEG = SEGS * CAP
    OFF_L, OFF_S, OFF_E, OFF_C = 0, TOTSEG, 2 * TOTSEG, 3 * TOTSEG
    BINLEN = 3 * TOTSEG + _NC * _NS * 16
    SHARE = BR // _NS
    assert E % _NS == 0 and EPT % S == 0 and S % 16 == 0 and SHARE % GC == 0
    assert NB % _NC == 0 and NBC <= 16

    mesh = plsc.VectorSubcoreMesh(
        core_axis_name="c", subcore_axis_name="s", num_cores=_NC, num_subcores=_NS
    )

    @functools.partial(
        pl.kernel,
        out_type=jax.ShapeDtypeStruct((BINLEN,), jnp.int32),
        mesh=mesh,
        scratch_types=[
            pltpu.VMEM((S,), jnp.int32),
            pltpu.VMEM((S,), jnp.int32),
            pltpu.VMEM((S,), jnp.int32),
            pltpu.VMEM((CAP,), jnp.int32),
            pltpu.VMEM((CAP,), jnp.int32),
            pltpu.VMEM((CAP,), jnp.int32),
            pltpu.VMEM((16,), jnp.int32),
            pltpu.SemaphoreType.DMA,
            pltpu.SemaphoreType.DMA,
        ],
        compiler_params=pltpu.CompilerParams(needs_layout_passes=False),
    )
    def sc_bin(src_hbm, dst_hbm, rel_hbm, out_hbm,
               dstv, relv, srcv, cl, cs, ce, cntv, si, sf):
        cid = lax.axis_index("c")
        sid = lax.axis_index("s")
        ebase = sid * EPT

        zero16i = jnp.zeros((16,), jnp.int32)
        trash16 = jnp.full((16,), TRASH, jnp.int32)
        iota16 = lax.iota(jnp.int32, 16)

        cntv[pl.ds(0, 16)] = zero16i

        @pl.loop(0, NBC)
        def _(bi):
            b = bi * _NC + cid
            lo = b * BR

            def chunk_body(ch, cnt):
                cbase = ebase + ch * S
                ld = pltpu.async_copy(dst_hbm.at[pl.ds(cbase, S)], dstv, si)
                lr = pltpu.async_copy(rel_hbm.at[pl.ds(cbase, S)], relv, si)
                ls = pltpu.async_copy(src_hbm.at[pl.ds(cbase, S)], srcv, si)
                ld.wait()
                lr.wait()
                ls.wait()

                def scan_body(v, cnt):
                    sl = pl.ds(v * 16, 16)
                    idx16 = dstv[sl] * R + relv[sl]
                    m = (idx16 >= lo) & (idx16 < lo + BR)
                    lidx16 = idx16 - lo
                    eid16 = cbase + v * 16 + iota16
                    mi = m.astype(jnp.int32)
                    pos16 = cnt + plsc.cumsum(mi) - 1
                    plsc.store_scatter(cl, [pos16], lidx16, mask=m)
                    plsc.store_scatter(cs, [pos16], srcv[sl], mask=m)
                    plsc.store_scatter(ce, [pos16], eid16, mask=m)
                    return cnt + jnp.sum(mi)

                return lax.fori_loop(0, S // 16, scan_body, cnt)

            cnt = lax.fori_loop(0, NCH, chunk_body, jnp.int32(0))

            @pl.loop(0, GC + 16, step=16)
            def _(i):
                cl[pl.ds(cnt + i, 16)] = trash16
                cs[pl.ds(cnt + i, 16)] = zero16i
                ce[pl.ds(cnt + i, 16)] = zero16i

            nfl = (cnt + GC - 1) // GC
            base = ((cid * _NS + sid) * NBC + bi) * CAP

            def flush_body(k, _):
                o = k * GC
                f1 = pltpu.async_copy(
                    cl.at[pl.ds(o, GC)], out_hbm.at[pl.ds(OFF_L + base + o, GC)], sf)
                f2 = pltpu.async_copy(
                    cs.at[pl.ds(o, GC)], out_hbm.at[pl.ds(OFF_S + base + o, GC)], sf)
                f3 = pltpu.async_copy(
                    ce.at[pl.ds(o, GC)], out_hbm.at[pl.ds(OFF_E + base + o, GC)], sf)
                f1.wait()
                f2.wait()
                f3.wait()
                return 0

            lax.fori_loop(0, nfl, flush_body, 0)
            plsc.store_scatter(
                cntv, [iota16], jnp.full((16,), nfl, jnp.int32), mask=(iota16 == bi))

        fc = pltpu.async_copy(
            cntv, out_hbm.at[pl.ds(OFF_C + (cid * _NS + sid) * 16, 16)], sf)
        fc.wait()

    @functools.partial(
        pl.kernel,
        out_type=jax.ShapeDtypeStruct((NB * BR, D), jnp.float32),
        mesh=mesh,
        scratch_types=[
            pltpu.VMEM((GC,), jnp.int32),
            pltpu.VMEM((GC,), jnp.int32),
            pltpu.VMEM((GC,), jnp.int32),
            pltpu.VMEM((GC,), jnp.int32),
            pltpu.VMEM((GC,), jnp.int32),
            pltpu.VMEM((GC,), jnp.int32),
            pltpu.VMEM((GC, D), jnp.float32),
            pltpu.VMEM((GC, D), jnp.float32),
            pltpu.VMEM((GC, D), jnp.float32),
            pltpu.VMEM((GC, D), jnp.float32),
            pltpu.VMEM((GC, D), jnp.float32),
            pltpu.VMEM((32,), jnp.int32),
            pltpu.VMEM_SHARED((BR + 8, D), jnp.float32),
            pltpu.SemaphoreType.DMA,
            pltpu.SemaphoreType.DMA,
            pltpu.SemaphoreType.DMA,
            pltpu.SemaphoreType.DMA,
            pltpu.SemaphoreType.DMA,
            pltpu.SemaphoreType.DMA,
        ],
        compiler_params=pltpu.CompilerParams(needs_layout_passes=False),
    )
    def sc_consume(hv_hbm, he_hbm, bin_hbm, out_hbm,
                   il0, is0, ie0, il1, is1, ie1, ra0, rb0, ra1, rb1, zb, cntv,
                   acc, si0, si1, sg0, sg1, ss0, ss1):
        cid = lax.axis_index("c")
        sid = lax.axis_index("s")

        zero16f = jnp.zeros((16,), jnp.float32)

        cntv[pl.ds(16, 16)] = jnp.zeros((16,), jnp.int32)
        lc = pltpu.async_copy(
            bin_hbm.at[pl.ds(OFF_C + (cid * _NS + sid) * 16, 16)],
            cntv.at[pl.ds(0, 16)], si0)

        @pl.loop(0, GC)
        def _(r):
            for c in range(D // 16):
                zb[r, pl.ds(c * 16, 16)] = zero16f

        lc.wait()

        @pl.loop(0, NBC)
        def _(bi):
            b = bi * _NC + cid
            lo = b * BR

            for z in range(SHARE // GC):
                pltpu.sync_copy(zb, acc.at[pl.ds(sid * SHARE + z * GC, GC)])

            @pl.when(sid == 0)
            def _():
                pltpu.sync_copy(zb.at[pl.ds(0, 8)], acc.at[pl.ds(BR, 8)])

            plsc.subcore_barrier()

            n128 = cntv[pl.ds(bi, 16)][0]
            base = ((cid * _NS + sid) * NBC + bi) * CAP
            npairs = n128 // 2

            def pair_body(k, _):
                o0 = base + k * (2 * GC)
                o1 = o0 + GC
                i0a = pltpu.async_copy(bin_hbm.at[pl.ds(OFF_L + o0, GC)], il0, si0)
                i0b = pltpu.async_copy(bin_hbm.at[pl.ds(OFF_S + o0, GC)], is0, si0)
                i0c = pltpu.async_copy(bin_hbm.at[pl.ds(OFF_E + o0, GC)], ie0, si0)
                i1a = pltpu.async_copy(bin_hbm.at[pl.ds(OFF_L + o1, GC)], il1, si1)
                i1b = pltpu.async_copy(bin_hbm.at[pl.ds(OFF_S + o1, GC)], is1, si1)
                i1c = pltpu.async_copy(bin_hbm.at[pl.ds(OFF_E + o1, GC)], ie1, si1)
                i0a.wait()
                i0b.wait()
                i0c.wait()
                g0a = pltpu.async_copy(hv_hbm.at[is0], ra0, sg0)
                g0b = pltpu.async_copy(he_hbm.at[ie0], rb0, sg0)
                i1a.wait()
                i1b.wait()
                i1c.wait()
                g1a = pltpu.async_copy(hv_hbm.at[is1], ra1, sg1)
                g1b = pltpu.async_copy(he_hbm.at[ie1], rb1, sg1)
                g0a.wait()
                g0b.wait()
                s0a = pltpu.async_copy(ra0, acc.at[il0], ss0, add=True)
                s0b = pltpu.async_copy(rb0, acc.at[il0], ss0, add=True)
                g1a.wait()
                g1b.wait()
                s1a = pltpu.async_copy(ra1, acc.at[il1], ss1, add=True)
                s1b = pltpu.async_copy(rb1, acc.at[il1], ss1, add=True)
                s0a.wait()
                s0b.wait()
                s1a.wait()
                s1b.wait()
                return 0

            lax.fori_loop(0, npairs, pair_body, 0)

            @pl.when(n128 % 2 == 1)
            def _():
                o0 = base + npairs * (2 * GC)
                ia = pltpu.async_copy(bin_hbm.at[pl.ds(OFF_L + o0, GC)], il0, si0)
                ib = pltpu.async_copy(bin_hbm.at[pl.ds(OFF_S + o0, GC)], is0, si0)
                ic = pltpu.async_copy(bin_hbm.at[pl.ds(OFF_E + o0, GC)], ie0, si0)
                ia.wait()
                ib.wait()
                ic.wait()
                ga = pltpu.async_copy(hv_hbm.at[is0], ra0, sg0)
                gb = pltpu.async_copy(he_hbm.at[ie0], rb0, sg0)
                ga.wait()
                gb.wait()
                sa = pltpu.async_copy(ra0, acc.at[il0], ss0, add=True)
                sb = pltpu.async_copy(rb0, acc.at[il0], ss0, add=True)
                sa.wait()
                sb.wait()

            plsc.subcore_barrier()
            pltpu.sync_copy(
                acc.at[pl.ds(sid * SHARE, SHARE)],
                out_hbm.at[pl.ds(lo + sid * SHARE, SHARE)],
            )

    return sc_bin, sc_consume


def kernel(h_v, edge_index, h_e, W_in, b_in, W_edge, b_edge, W_lin, W_out):
    N, D = h_v.shape
    E, DE = h_e.shape
    L = W_in.shape[0]
    R = W_lin.shape[1] // D
    NR = N * R

    sc_bin, sc_consume = _make_sc_kernels(N, E, D, R)
    ei = edge_index.astype(jnp.int32)
    src, dst, rel = ei[0], ei[1], ei[2]

    binfo = sc_bin(src, dst, rel)

    h = h_v
    for l in range(L):
        hv = _tc_in_mlp(h, W_in[l], b_in[l])
        he = _tc_edge_mlp(h_e, W_edge[l], b_edge[l])
        upd_full = sc_consume(hv, he, binfo)
        upd = upd_full[:NR].reshape(N, R * D)
        h = _tc_out_mlp(upd, h, W_lin[l], W_out[l])
    return h
